# baseline (speedup 1.0000x reference)
import jax
import jax.numpy as jnp
from jax.experimental import pallas as pl
from jax.experimental.pallas import tpu as pltpu

_THR = 0.2
_FILL = -1000000000.0
_GONE = -2000000000.0
_K = 20


def _nms_topk_kernel(x_ref, out_ref, m_ref):
    cpb, h, w = x_ref.shape
    x = x_ref[...]
    neg = jnp.float32(-jnp.inf)
    colpad = jnp.full((cpb, h, 1), neg, jnp.float32)
    left = jnp.concatenate([colpad, x[:, :, :-1]], axis=2)
    right = jnp.concatenate([x[:, :, 1:], colpad], axis=2)
    hmax = jnp.maximum(left, right)
    h3 = jnp.maximum(hmax, x)
    rowpad = jnp.full((cpb, 1, w), neg, jnp.float32)
    above = jnp.concatenate([rowpad, h3[:, :-1, :]], axis=1)
    below = jnp.concatenate([h3[:, 1:, :], rowpad], axis=1)
    nmax = jnp.maximum(hmax, jnp.maximum(above, below))
    masked = jnp.where((x > nmax) & (x > _THR), x, jnp.float32(_FILL))
    m_ref[...] = masked

    rm_all = jnp.max(masked.reshape(cpb, h // 8, 8, w), axis=3)
    riota = (jax.lax.broadcasted_iota(jnp.int32, (h // 8, 8), 0) * 8
             + jax.lax.broadcasted_iota(jnp.int32, (h // 8, 8), 1))
    ciota = jax.lax.broadcasted_iota(jnp.int32, (1, w), 1)
    big = jnp.int32(1 << 30)

    rowmaxes = [rm_all[p] for p in range(cpb)]
    vals = [[] for _ in range(cpb)]
    xs = [[] for _ in range(cpb)]
    ys = [[] for _ in range(cpb)]
    for _ in range(_K):
        for p in range(cpb):
            rm = rowmaxes[p]
            m = jnp.max(rm)
            r = jnp.min(jnp.where(rm == m, riota, big))
            row = m_ref[p, pl.ds(r, 1), :]
            c = jnp.min(jnp.where(row == m, ciota, big))
            vals[p].append(m)
            xs[p].append(c.astype(jnp.float32))
            ys[p].append(r.astype(jnp.float32))
            new_row = jnp.where(ciota == c, jnp.float32(_GONE), row)
            m_ref[p, pl.ds(r, 1), :] = new_row
            rowmaxes[p] = jnp.where(riota == r, jnp.max(new_row), rm)
    for p in range(cpb):
        out_ref[p, 0, 0:_K] = jnp.stack(vals[p])
        out_ref[p, 1, 0:_K] = jnp.stack(xs[p])
        out_ref[p, 2, 0:_K] = jnp.stack(ys[p])


def kernel(confmaps, k):
    b, n, h, w = confmaps.shape
    planes = b * n
    cpb = 8 if planes % 8 == 0 else 1
    x = confmaps.reshape(planes, h, w)
    out = pl.pallas_call(
        _nms_topk_kernel,
        grid=(planes // cpb,),
        in_specs=[pl.BlockSpec((cpb, h, w), lambda i: (i, 0, 0))],
        out_specs=pl.BlockSpec((cpb, 8, 128), lambda i: (i, 0, 0)),
        out_shape=jax.ShapeDtypeStruct((planes, 8, 128), jnp.float32),
        scratch_shapes=[pltpu.VMEM((cpb, h, w), jnp.float32)],
    )(x)
    vals = out[:, 0, :_K].reshape(b, n, _K)
    xcoord = out[:, 1, :_K].reshape(b, n, _K)
    ycoord = out[:, 2, :_K].reshape(b, n, _K)
    peaks = jnp.stack([xcoord, ycoord], axis=-1)
    valid = vals > jnp.float32(_THR)
    return peaks, vals, valid


# speedup vs baseline: 1.3934x; 1.3934x over previous
import jax
import jax.numpy as jnp
from jax.experimental import pallas as pl
from jax.experimental.pallas import tpu as pltpu

_THR = 0.2
_FILL = -1000000000.0
_GONE = -2000000000.0
_K = 20
_RPC = 8


def _nms_topk_kernel(x_ref, out_ref, *m_refs):
    cpb, h, w = x_ref.shape
    nch = h // _RPC
    neg = jnp.float32(-jnp.inf)

    chunkmaxes = []
    for p in range(cpb):
        x = x_ref[p]
        colpad = jnp.full((h, 1), neg, jnp.float32)
        left = jnp.concatenate([colpad, x[:, :-1]], axis=1)
        right = jnp.concatenate([x[:, 1:], colpad], axis=1)
        hmax = jnp.maximum(left, right)
        h3 = jnp.maximum(hmax, x)
        rowpad = jnp.full((1, w), neg, jnp.float32)
        above = jnp.concatenate([rowpad, h3[:-1, :]], axis=0)
        below = jnp.concatenate([h3[1:, :], rowpad], axis=0)
        nmax = jnp.maximum(hmax, jnp.maximum(above, below))
        masked = jnp.where((x > nmax) & (x > _THR), x, jnp.float32(_FILL))
        m_refs[p][...] = masked
        cm = jnp.max(masked.reshape(nch, _RPC, w), axis=(1, 2), keepdims=True)
        chunkmaxes.append(cm)

    chiota = jax.lax.broadcasted_iota(jnp.int32, (nch, 1, 1), 0)
    liota = (jax.lax.broadcasted_iota(jnp.int32, (_RPC, w), 0) * w
             + jax.lax.broadcasted_iota(jnp.int32, (_RPC, w), 1))
    big = jnp.int32(1 << 30)
    vals = [[] for _ in range(cpb)]
    xs = [[] for _ in range(cpb)]
    ys = [[] for _ in range(cpb)]
    for _ in range(_K):
        for p in range(cpb):
            cm = chunkmaxes[p]
            m = jnp.max(cm)
            ch = jnp.min(jnp.where(cm == m, chiota, big))
            chunk = m_refs[p][pl.ds(ch * _RPC, _RPC), :]
            fl = jnp.min(jnp.where(chunk == m, liota, big))
            f = ch * (_RPC * w) + fl
            vals[p].append(m)
            xs[p].append((f % w).astype(jnp.float32))
            ys[p].append((f // w).astype(jnp.float32))
            new_chunk = jnp.where(liota == fl, jnp.float32(_GONE), chunk)
            m_refs[p][pl.ds(ch * _RPC, _RPC), :] = new_chunk
            chunkmaxes[p] = jnp.where(chiota == ch, jnp.max(new_chunk), cm)
    for p in range(cpb):
        out_ref[p, 0, 0:_K] = jnp.stack(vals[p])
        out_ref[p, 1, 0:_K] = jnp.stack(xs[p])
        out_ref[p, 2, 0:_K] = jnp.stack(ys[p])


def kernel(confmaps, k):
    b, n, h, w = confmaps.shape
    planes = b * n
    cpb = 4 if planes % 4 == 0 else 1
    x = confmaps.reshape(planes, h, w)
    out = pl.pallas_call(
        _nms_topk_kernel,
        grid=(planes // cpb,),
        in_specs=[pl.BlockSpec((cpb, h, w), lambda i: (i, 0, 0))],
        out_specs=pl.BlockSpec((cpb, 8, 128), lambda i: (i, 0, 0)),
        out_shape=jax.ShapeDtypeStruct((planes, 8, 128), jnp.float32),
        scratch_shapes=[pltpu.VMEM((h, w), jnp.float32) for _ in range(cpb)],
    )(x)
    vals = out[:, 0, :_K].reshape(b, n, _K)
    xcoord = out[:, 1, :_K].reshape(b, n, _K)
    ycoord = out[:, 2, :_K].reshape(b, n, _K)
    peaks = jnp.stack([xcoord, ycoord], axis=-1)
    valid = vals > jnp.float32(_THR)
    return peaks, vals, valid


# speedup vs baseline: 1.6333x; 1.1722x over previous
import jax
import jax.numpy as jnp
from jax.experimental import pallas as pl
from jax.experimental.pallas import tpu as pltpu

_THR = 0.2
_FILL = -1000000000.0
_GONE = -2000000000.0
_K = 20
_RPC = 8


def _nms_topk_kernel(x_ref, out_ref, *m_refs):
    cpb, h, w = x_ref.shape
    nch = h // _RPC
    neg = jnp.float32(-jnp.inf)

    chunkmaxes = []
    for p in range(cpb):
        x = x_ref[p]
        colpad = jnp.full((h, 1), neg, jnp.float32)
        left = jnp.concatenate([colpad, x[:, :-1]], axis=1)
        right = jnp.concatenate([x[:, 1:], colpad], axis=1)
        hmax = jnp.maximum(left, right)
        h3 = jnp.maximum(hmax, x)
        rowpad = jnp.full((1, w), neg, jnp.float32)
        above = jnp.concatenate([rowpad, h3[:-1, :]], axis=0)
        below = jnp.concatenate([h3[1:, :], rowpad], axis=0)
        nmax = jnp.maximum(hmax, jnp.maximum(above, below))
        masked = jnp.where((x > nmax) & (x > _THR), x, jnp.float32(_FILL))
        m_refs[p][...] = masked
        cm = jnp.max(masked.reshape(nch, _RPC, w), axis=(1, 2))
        chunkmaxes.append(cm.reshape(1, nch))

    chiota = jax.lax.broadcasted_iota(jnp.int32, (1, nch), 1)
    liota = (jax.lax.broadcasted_iota(jnp.int32, (_RPC, w), 0) * 512
             + jax.lax.broadcasted_iota(jnp.int32, (_RPC, w), 1))
    oiota = jax.lax.broadcasted_iota(jnp.int32, (1, 128), 1)
    big = jnp.int32(1 << 30)
    vvec = [jnp.zeros((1, 128), jnp.float32) for _ in range(cpb)]
    xvec = [jnp.zeros((1, 128), jnp.float32) for _ in range(cpb)]
    yvec = [jnp.zeros((1, 128), jnp.float32) for _ in range(cpb)]
    for i in range(_K):
        for p in range(cpb):
            cm = chunkmaxes[p]
            m = jnp.max(cm, axis=1, keepdims=True)
            cmask = cm == m
            ch_v = jnp.min(jnp.where(cmask, chiota, big), axis=1, keepdims=True)
            ch_s = jnp.min(jnp.where(cmask, chiota, big))
            chunk = m_refs[p][pl.ds(ch_s * _RPC, _RPC), :]
            fl = jnp.min(jnp.where(chunk == m, liota, big),
                         axis=(0, 1), keepdims=True)
            col = jnp.bitwise_and(fl, 511)
            row = (ch_v * _RPC + (fl >> 9))
            vvec[p] = jnp.where(oiota == i, m, vvec[p])
            xvec[p] = jnp.where(oiota == i, col.astype(jnp.float32), xvec[p])
            yvec[p] = jnp.where(oiota == i, row.astype(jnp.float32), yvec[p])
            new_chunk = jnp.where(liota == fl, jnp.float32(_GONE), chunk)
            m_refs[p][pl.ds(ch_s * _RPC, _RPC), :] = new_chunk
            newmax = jnp.max(new_chunk, axis=(0, 1), keepdims=True)
            chunkmaxes[p] = jnp.where(chiota == ch_v, newmax, cm)
    for p in range(cpb):
        out_ref[p, 0:1, :] = vvec[p]
        out_ref[p, 1:2, :] = xvec[p]
        out_ref[p, 2:3, :] = yvec[p]


def kernel(confmaps, k):
    b, n, h, w = confmaps.shape
    planes = b * n
    cpb = 4 if planes % 4 == 0 else 1
    x = confmaps.reshape(planes, h, w)
    out = pl.pallas_call(
        _nms_topk_kernel,
        grid=(planes // cpb,),
        in_specs=[pl.BlockSpec((cpb, h, w), lambda i: (i, 0, 0))],
        out_specs=pl.BlockSpec((cpb, 8, 128), lambda i: (i, 0, 0)),
        out_shape=jax.ShapeDtypeStruct((planes, 8, 128), jnp.float32),
        scratch_shapes=[pltpu.VMEM((h, w), jnp.float32) for _ in range(cpb)],
    )(x)
    vals = out[:, 0, :_K].reshape(b, n, _K)
    xcoord = out[:, 1, :_K].reshape(b, n, _K)
    ycoord = out[:, 2, :_K].reshape(b, n, _K)
    peaks = jnp.stack([xcoord, ycoord], axis=-1)
    valid = vals > jnp.float32(_THR)
    return peaks, vals, valid


# speedup vs baseline: 4.9625x; 3.0383x over previous
import jax
import jax.numpy as jnp
from jax.experimental import pallas as pl
from jax.experimental.pallas import tpu as pltpu

_THR = 0.2
_FILL = -1000000000.0
_K = 20
_RPC = 8


def _nms_topk_kernel(x_ref, out_ref, *m_refs):
    cpb, h, w = x_ref.shape
    nch = h // _RPC
    neg = jnp.float32(-jnp.inf)

    cms = []
    for p in range(cpb):
        x = x_ref[p]
        colpad = jnp.full((h, 1), neg, jnp.float32)
        left = jnp.concatenate([colpad, x[:, :-1]], axis=1)
        right = jnp.concatenate([x[:, 1:], colpad], axis=1)
        hmax = jnp.maximum(left, right)
        h3 = jnp.maximum(hmax, x)
        rowpad = jnp.full((1, w), neg, jnp.float32)
        above = jnp.concatenate([rowpad, h3[:-1, :]], axis=0)
        below = jnp.concatenate([h3[1:, :], rowpad], axis=0)
        nmax = jnp.maximum(hmax, jnp.maximum(above, below))
        masked = jnp.where((x > nmax) & (x > _THR), x, jnp.float32(_FILL))
        m_refs[p][...] = masked
        cm = jnp.max(masked.reshape(nch, _RPC, w), axis=(1, 2))
        cms.append(cm.reshape(1, nch))

    cm8 = jnp.concatenate(cms, axis=0)
    chiota = jax.lax.broadcasted_iota(jnp.int32, (cpb, nch), 1)
    liota3 = (jax.lax.broadcasted_iota(jnp.int32, (1, _RPC, w), 1) * 512
              + jax.lax.broadcasted_iota(jnp.int32, (1, _RPC, w), 2))
    oiota = jax.lax.broadcasted_iota(jnp.int32, (cpb, 128), 1)
    big = jnp.int32(1 << 30)
    vvec = jnp.zeros((cpb, 128), jnp.float32)
    xvec = jnp.zeros((cpb, 128), jnp.float32)
    yvec = jnp.zeros((cpb, 128), jnp.float32)
    vlast = jnp.full((cpb, nch), jnp.inf, jnp.float32)
    fllast = jnp.full((cpb, nch), -1, jnp.int32)
    for i in range(_K):
        m = jnp.max(cm8, axis=1, keepdims=True)
        cmask = cm8 == m
        ch_v = jnp.min(jnp.where(cmask, chiota, big), axis=1, keepdims=True)
        chunk8 = jnp.concatenate(
            [m_refs[p][pl.ds(ch_v[p, 0] * _RPC, _RPC), :][None]
             for p in range(cpb)], axis=0)
        sel = chiota == ch_v
        vl = jnp.max(jnp.where(sel, vlast, neg), axis=1, keepdims=True)
        fll = jnp.max(jnp.where(sel, fllast, -2), axis=1, keepdims=True)
        vl3 = vl[:, :, None]
        fll3 = fll[:, :, None]
        m3 = m[:, :, None]
        elig = (chunk8 < vl3) | ((chunk8 == vl3) & (liota3 > fll3))
        fl3 = jnp.min(jnp.where(elig & (chunk8 == m3), liota3, big),
                      axis=(1, 2), keepdims=True)
        fl = fl3[:, :, 0]
        col = jnp.bitwise_and(fl, 511)
        row = ch_v * _RPC + (fl >> 9)
        vvec = jnp.where(oiota == i, m, vvec)
        xvec = jnp.where(oiota == i, col.astype(jnp.float32), xvec)
        yvec = jnp.where(oiota == i, row.astype(jnp.float32), yvec)
        e2 = (chunk8 < m3) | ((chunk8 == m3) & (liota3 > fl3))
        newmax = jnp.max(jnp.where(e2, chunk8, neg), axis=(1, 2),
                         keepdims=True)[:, :, 0]
        cm8 = jnp.where(sel, newmax, cm8)
        vlast = jnp.where(sel, m, vlast)
        fllast = jnp.where(sel, fl, fllast)
    out_ref[:, 0, :] = vvec
    out_ref[:, 1, :] = xvec
    out_ref[:, 2, :] = yvec


def kernel(confmaps, k):
    b, n, h, w = confmaps.shape
    planes = b * n
    cpb = 8 if planes % 8 == 0 else 1
    x = confmaps.reshape(planes, h, w)
    out = pl.pallas_call(
        _nms_topk_kernel,
        grid=(planes // cpb,),
        in_specs=[pl.BlockSpec((cpb, h, w), lambda i: (i, 0, 0))],
        out_specs=pl.BlockSpec((cpb, 8, 128), lambda i: (i, 0, 0)),
        out_shape=jax.ShapeDtypeStruct((planes, 8, 128), jnp.float32),
        scratch_shapes=[pltpu.VMEM((h, w), jnp.float32) for _ in range(cpb)],
    )(x)
    vals = out[:, 0, :_K].reshape(b, n, _K)
    xcoord = out[:, 1, :_K].reshape(b, n, _K)
    ycoord = out[:, 2, :_K].reshape(b, n, _K)
    peaks = jnp.stack([xcoord, ycoord], axis=-1)
    valid = vals > jnp.float32(_THR)
    return peaks, vals, valid


# speedup vs baseline: 8.9023x; 1.7939x over previous
import jax
import jax.numpy as jnp
from jax.experimental import pallas as pl
from jax.experimental.pallas import tpu as pltpu

_THR = 0.2
_FILL = -1000000000.0
_K = 20
_RPC = 8


def _nms_topk_kernel(x_ref, out_ref, *m_refs):
    cpb, h, w = x_ref.shape
    nch = h // _RPC
    neg = jnp.float32(-jnp.inf)

    cms = []
    for p in range(cpb):
        x = x_ref[p]
        colpad = jnp.full((h, 1), neg, jnp.float32)
        left = jnp.concatenate([colpad, x[:, :-1]], axis=1)
        right = jnp.concatenate([x[:, 1:], colpad], axis=1)
        hmax = jnp.maximum(left, right)
        h3 = jnp.maximum(hmax, x)
        rowpad = jnp.full((1, w), neg, jnp.float32)
        above = jnp.concatenate([rowpad, h3[:-1, :]], axis=0)
        below = jnp.concatenate([h3[1:, :], rowpad], axis=0)
        nmax = jnp.maximum(hmax, jnp.maximum(above, below))
        masked = jnp.where((x > nmax) & (x > _THR), x, jnp.float32(_FILL))
        m_refs[p][...] = masked
        cm = jnp.max(masked.reshape(nch, _RPC, w), axis=(1, 2))
        cms.append(cm.reshape(1, nch))

    cm8 = jnp.concatenate(cms, axis=0)
    chiota = jax.lax.broadcasted_iota(jnp.int32, (cpb, nch), 1)
    liota3 = (jax.lax.broadcasted_iota(jnp.int32, (1, _RPC, w), 1) * 512
              + jax.lax.broadcasted_iota(jnp.int32, (1, _RPC, w), 2))
    oiota = jax.lax.broadcasted_iota(jnp.int32, (cpb, 128), 1)
    big = jnp.int32(1 << 30)
    vvec = jnp.zeros((cpb, 128), jnp.float32)
    xvec = jnp.zeros((cpb, 128), jnp.float32)
    yvec = jnp.zeros((cpb, 128), jnp.float32)
    vlast = jnp.full((cpb, nch), jnp.inf, jnp.float32)
    fllast = jnp.full((cpb, nch), -1, jnp.int32)
    for i in range(_K):
        m = jnp.max(cm8, axis=1, keepdims=True)
        cmask = cm8 == m
        ch_v = jnp.min(jnp.where(cmask, chiota, big), axis=1, keepdims=True)
        chunk8 = jnp.concatenate(
            [m_refs[p][pl.ds(ch_v[p, 0] * _RPC, _RPC), :][None]
             for p in range(cpb)], axis=0)
        sel = chiota == ch_v
        vl = jnp.max(jnp.where(sel, vlast, neg), axis=1, keepdims=True)
        fll = jnp.max(jnp.where(sel, fllast, -2), axis=1, keepdims=True)
        vl3 = vl[:, :, None]
        fll3 = fll[:, :, None]
        m3 = m[:, :, None]
        elig = (chunk8 < vl3) | ((chunk8 == vl3) & (liota3 > fll3))
        fl3 = jnp.min(jnp.where(elig & (chunk8 == m3), liota3, big),
                      axis=(1, 2), keepdims=True)
        fl = fl3[:, :, 0]
        col = jnp.bitwise_and(fl, 511)
        row = ch_v * _RPC + (fl >> 9)
        vvec = jnp.where(oiota == i, m, vvec)
        xvec = jnp.where(oiota == i, col.astype(jnp.float32), xvec)
        yvec = jnp.where(oiota == i, row.astype(jnp.float32), yvec)
        e2 = (chunk8 < m3) | ((chunk8 == m3) & (liota3 > fl3))
        newmax = jnp.max(jnp.where(e2, chunk8, neg), axis=(1, 2),
                         keepdims=True)[:, :, 0]
        cm8 = jnp.where(sel, newmax, cm8)
        vlast = jnp.where(sel, m, vlast)
        fllast = jnp.where(sel, fl, fllast)
    out_ref[:, 0, :] = vvec
    out_ref[:, 1, :] = xvec
    out_ref[:, 2, :] = yvec


def kernel(confmaps, k):
    b, n, h, w = confmaps.shape
    planes = b * n
    cpb = 24 if planes % 24 == 0 else 1
    x = confmaps.reshape(planes, h, w)
    out = pl.pallas_call(
        _nms_topk_kernel,
        grid=(planes // cpb,),
        in_specs=[pl.BlockSpec((cpb, h, w), lambda i: (i, 0, 0))],
        out_specs=pl.BlockSpec((cpb, 8, 128), lambda i: (i, 0, 0)),
        out_shape=jax.ShapeDtypeStruct((planes, 8, 128), jnp.float32),
        scratch_shapes=[pltpu.VMEM((h, w), jnp.float32) for _ in range(cpb)],
    )(x)
    vals = out[:, 0, :_K].reshape(b, n, _K)
    xcoord = out[:, 1, :_K].reshape(b, n, _K)
    ycoord = out[:, 2, :_K].reshape(b, n, _K)
    peaks = jnp.stack([xcoord, ycoord], axis=-1)
    valid = vals > jnp.float32(_THR)
    return peaks, vals, valid


# speedup vs baseline: 9.0011x; 1.0111x over previous
import jax
import jax.numpy as jnp
from jax.experimental import pallas as pl
from jax.experimental.pallas import tpu as pltpu

_THR = 0.2
_FILL = -1000000000.0
_K = 20
_RPC = 8


def _nms_topk_kernel(x_ref, out_ref, *m_refs):
    cpb, h, w = x_ref.shape
    nch = h // _RPC
    neg = jnp.float32(-jnp.inf)

    cms = []
    for p in range(cpb):
        x = x_ref[p]
        colpad = jnp.full((h, 1), neg, jnp.float32)
        left = jnp.concatenate([colpad, x[:, :-1]], axis=1)
        right = jnp.concatenate([x[:, 1:], colpad], axis=1)
        hmax = jnp.maximum(left, right)
        h3 = jnp.maximum(hmax, x)
        rowpad = jnp.full((1, w), neg, jnp.float32)
        above = jnp.concatenate([rowpad, h3[:-1, :]], axis=0)
        below = jnp.concatenate([h3[1:, :], rowpad], axis=0)
        nmax = jnp.maximum(hmax, jnp.maximum(above, below))
        masked = jnp.where((x > nmax) & (x > _THR), x, jnp.float32(_FILL))
        m_refs[p][...] = masked
        cm = jnp.max(masked.reshape(nch, _RPC, w), axis=(1, 2))
        cms.append(cm.reshape(1, nch))

    cm8 = jnp.concatenate(cms, axis=0)
    chiota = jax.lax.broadcasted_iota(jnp.int32, (cpb, nch), 1)
    liota3 = (jax.lax.broadcasted_iota(jnp.int32, (1, _RPC, w), 1) * 512
              + jax.lax.broadcasted_iota(jnp.int32, (1, _RPC, w), 2))
    oiota = jax.lax.broadcasted_iota(jnp.int32, (cpb, 128), 1)
    big = jnp.int32(1 << 30)
    vvec = jnp.zeros((cpb, 128), jnp.float32)
    xvec = jnp.zeros((cpb, 128), jnp.float32)
    yvec = jnp.zeros((cpb, 128), jnp.float32)
    vlast = jnp.full((cpb, nch), jnp.inf, jnp.float32)
    fllast = jnp.full((cpb, nch), -1, jnp.int32)
    for i in range(_K):
        m = jnp.max(cm8, axis=1, keepdims=True)
        cmask = cm8 == m
        ch_v = jnp.min(jnp.where(cmask, chiota, big), axis=1, keepdims=True)
        chunk8 = jnp.concatenate(
            [m_refs[p][pl.ds(ch_v[p, 0] * _RPC, _RPC), :][None]
             for p in range(cpb)], axis=0)
        sel = chiota == ch_v
        vl = jnp.max(jnp.where(sel, vlast, neg), axis=1, keepdims=True)
        fll = jnp.max(jnp.where(sel, fllast, -2), axis=1, keepdims=True)
        vl3 = vl[:, :, None]
        fll3 = fll[:, :, None]
        m3 = m[:, :, None]
        elig = (chunk8 < vl3) | ((chunk8 == vl3) & (liota3 > fll3))
        fl3 = jnp.min(jnp.where(elig & (chunk8 == m3), liota3, big),
                      axis=(1, 2), keepdims=True)
        fl = fl3[:, :, 0]
        col = jnp.bitwise_and(fl, 511)
        row = ch_v * _RPC + (fl >> 9)
        vvec = jnp.where(oiota == i, m, vvec)
        xvec = jnp.where(oiota == i, col.astype(jnp.float32), xvec)
        yvec = jnp.where(oiota == i, row.astype(jnp.float32), yvec)
        e2 = elig & (liota3 != fl3)
        newmax = jnp.max(jnp.where(e2, chunk8, neg), axis=(1, 2),
                         keepdims=True)[:, :, 0]
        cm8 = jnp.where(sel, newmax, cm8)
        vlast = jnp.where(sel, m, vlast)
        fllast = jnp.where(sel, fl, fllast)
    out_ref[:, 0, :] = vvec
    out_ref[:, 1, :] = xvec
    out_ref[:, 2, :] = yvec


def kernel(confmaps, k):
    b, n, h, w = confmaps.shape
    planes = b * n
    cpb = 24 if planes % 24 == 0 else 1
    x = confmaps.reshape(planes, h, w)
    out = pl.pallas_call(
        _nms_topk_kernel,
        grid=(planes // cpb,),
        in_specs=[pl.BlockSpec((cpb, h, w), lambda i: (i, 0, 0))],
        out_specs=pl.BlockSpec((cpb, 8, 128), lambda i: (i, 0, 0)),
        out_shape=jax.ShapeDtypeStruct((planes, 8, 128), jnp.float32),
        scratch_shapes=[pltpu.VMEM((h, w), jnp.float32) for _ in range(cpb)],
    )(x)
    vals = out[:, 0, :_K].reshape(b, n, _K)
    xcoord = out[:, 1, :_K].reshape(b, n, _K)
    ycoord = out[:, 2, :_K].reshape(b, n, _K)
    peaks = jnp.stack([xcoord, ycoord], axis=-1)
    valid = vals > jnp.float32(_THR)
    return peaks, vals, valid


# speedup vs baseline: 9.8868x; 1.0984x over previous
import jax
import jax.numpy as jnp
from jax.experimental import pallas as pl
from jax.experimental.pallas import tpu as pltpu

_THR = 0.2
_FILL = -1000000000.0
_K = 20
_RPC = 8


def _nms_topk_kernel(x_ref, out_ref):
    cpb, h, w = x_ref.shape
    nch = h // _RPC
    neg = jnp.float32(-jnp.inf)

    cms = []
    for p in range(cpb):
        x = x_ref[p]
        colpad = jnp.full((h, 1), neg, jnp.float32)
        left = jnp.concatenate([colpad, x[:, :-1]], axis=1)
        right = jnp.concatenate([x[:, 1:], colpad], axis=1)
        hmax = jnp.maximum(left, right)
        h3 = jnp.maximum(hmax, x)
        rowpad = jnp.full((1, w), neg, jnp.float32)
        above = jnp.concatenate([rowpad, h3[:-1, :]], axis=0)
        below = jnp.concatenate([h3[1:, :], rowpad], axis=0)
        nmax = jnp.maximum(jnp.maximum(hmax, jnp.float32(_THR)),
                           jnp.maximum(above, below))
        masked = jnp.where(x > nmax, x, jnp.float32(_FILL))
        x_ref[p] = masked
        cm = jnp.max(masked.reshape(nch, _RPC, w), axis=(1, 2))
        cms.append(cm.reshape(1, nch))

    cm8 = jnp.concatenate(cms, axis=0)
    chiota = jax.lax.broadcasted_iota(jnp.int32, (cpb, nch), 1)
    liota3 = (jax.lax.broadcasted_iota(jnp.int32, (1, _RPC, w), 1) * 512
              + jax.lax.broadcasted_iota(jnp.int32, (1, _RPC, w), 2))
    oiota = jax.lax.broadcasted_iota(jnp.int32, (cpb, 128), 1)
    big = jnp.int32(1 << 30)
    vvec = jnp.zeros((cpb, 128), jnp.float32)
    xvec = jnp.zeros((cpb, 128), jnp.float32)
    yvec = jnp.zeros((cpb, 128), jnp.float32)
    vlast = jnp.full((cpb, nch), jnp.inf, jnp.float32)
    fllast = jnp.full((cpb, nch), -1, jnp.int32)
    for i in range(_K):
        m = jnp.max(cm8, axis=1, keepdims=True)
        cmask = cm8 == m
        ch_v = jnp.min(jnp.where(cmask, chiota, big), axis=1, keepdims=True)
        chunk8 = jnp.concatenate(
            [x_ref[p, pl.ds(ch_v[p, 0] * _RPC, _RPC), :][None]
             for p in range(cpb)], axis=0)
        sel = chiota == ch_v
        vl = jnp.max(jnp.where(sel, vlast, neg), axis=1, keepdims=True)
        fll = jnp.max(jnp.where(sel, fllast, -2), axis=1, keepdims=True)
        vl3 = vl[:, :, None]
        fll3 = fll[:, :, None]
        m3 = m[:, :, None]
        elig = (chunk8 < vl3) | ((chunk8 == vl3) & (liota3 > fll3))
        fl3 = jnp.min(jnp.where(elig & (chunk8 == m3), liota3, big),
                      axis=(1, 2), keepdims=True)
        fl = fl3[:, :, 0]
        col = jnp.bitwise_and(fl, 511)
        row = ch_v * _RPC + (fl >> 9)
        vvec = jnp.where(oiota == i, m, vvec)
        xvec = jnp.where(oiota == i, col.astype(jnp.float32), xvec)
        yvec = jnp.where(oiota == i, row.astype(jnp.float32), yvec)
        e2 = elig & (liota3 != fl3)
        newmax = jnp.max(jnp.where(e2, chunk8, neg), axis=(1, 2),
                         keepdims=True)[:, :, 0]
        cm8 = jnp.where(sel, newmax, cm8)
        vlast = jnp.where(sel, m, vlast)
        fllast = jnp.where(sel, fl, fllast)
    out_ref[:, 0, :] = vvec
    out_ref[:, 1, :] = xvec
    out_ref[:, 2, :] = yvec


def kernel(confmaps, k):
    b, n, h, w = confmaps.shape
    planes = b * n
    cpb = 32 if planes % 32 == 0 else 1
    x = confmaps.reshape(planes, h, w)
    out = pl.pallas_call(
        _nms_topk_kernel,
        grid=(planes // cpb,),
        in_specs=[pl.BlockSpec((cpb, h, w), lambda i: (i, 0, 0))],
        out_specs=pl.BlockSpec((cpb, 8, 128), lambda i: (i, 0, 0)),
        out_shape=jax.ShapeDtypeStruct((planes, 8, 128), jnp.float32),
    )(x)
    vals = out[:, 0, :_K].reshape(b, n, _K)
    xcoord = out[:, 1, :_K].reshape(b, n, _K)
    ycoord = out[:, 2, :_K].reshape(b, n, _K)
    peaks = jnp.stack([xcoord, ycoord], axis=-1)
    valid = vals > jnp.float32(_THR)
    return peaks, vals, valid


# speedup vs baseline: 11.0774x; 1.1204x over previous
import jax
import jax.numpy as jnp
from jax.experimental import pallas as pl

_THR = 0.2
_FILL = -1000000000.0
_GONE = -2000000000.0
_K = 20
_RPC = 8


def _nms_topk_kernel(x_ref, out_ref):
    cpb, h, w = x_ref.shape
    nch = h // _RPC
    neg = jnp.float32(-jnp.inf)

    cms = []
    for p in range(cpb):
        x = x_ref[p]
        colpad = jnp.full((h, 1), neg, jnp.float32)
        left = jnp.concatenate([colpad, x[:, :-1]], axis=1)
        right = jnp.concatenate([x[:, 1:], colpad], axis=1)
        hmax = jnp.maximum(left, right)
        h3 = jnp.maximum(hmax, x)
        rowpad = jnp.full((1, w), neg, jnp.float32)
        above = jnp.concatenate([rowpad, h3[:-1, :]], axis=0)
        below = jnp.concatenate([h3[1:, :], rowpad], axis=0)
        nmax = jnp.maximum(jnp.maximum(hmax, jnp.float32(_THR)),
                           jnp.maximum(above, below))
        masked = jnp.where(x > nmax, x, jnp.float32(_FILL))
        x_ref[p] = masked
        cm = jnp.max(masked.reshape(nch, _RPC, w), axis=(1, 2))
        cms.append(cm.reshape(1, nch))

    cm8 = jnp.concatenate(cms, axis=0)
    chiota = jax.lax.broadcasted_iota(jnp.int32, (cpb, nch), 1)
    liota3 = (jax.lax.broadcasted_iota(jnp.int32, (1, _RPC, w), 1) * 512
              + jax.lax.broadcasted_iota(jnp.int32, (1, _RPC, w), 2))
    oiota = jax.lax.broadcasted_iota(jnp.int32, (cpb, 128), 1)
    big = jnp.int32(1 << 30)
    vvec = jnp.zeros((cpb, 128), jnp.float32)
    xvec = jnp.zeros((cpb, 128), jnp.float32)
    yvec = jnp.zeros((cpb, 128), jnp.float32)
    for i in range(_K):
        m = jnp.max(cm8, axis=1, keepdims=True)
        cmask = cm8 == m
        ch_v = jnp.min(jnp.where(cmask, chiota, big), axis=1, keepdims=True)
        chunk8 = jnp.concatenate(
            [x_ref[p, pl.ds(ch_v[p, 0] * _RPC, _RPC), :][None]
             for p in range(cpb)], axis=0)
        sel = chiota == ch_v
        m3 = m[:, :, None]
        fl3 = jnp.min(jnp.where(chunk8 == m3, liota3, big),
                      axis=(1, 2), keepdims=True)
        fl = fl3[:, :, 0]
        col = jnp.bitwise_and(fl, 511)
        row = ch_v * _RPC + (fl >> 9)
        vvec = jnp.where(oiota == i, m, vvec)
        xvec = jnp.where(oiota == i, col.astype(jnp.float32), xvec)
        yvec = jnp.where(oiota == i, row.astype(jnp.float32), yvec)
        new8 = jnp.where(liota3 == fl3, jnp.float32(_GONE), chunk8)
        for p in range(cpb):
            x_ref[p, pl.ds(ch_v[p, 0] * _RPC, _RPC), :] = new8[p]
        newmax = jnp.max(new8, axis=(1, 2), keepdims=True)[:, :, 0]
        cm8 = jnp.where(sel, newmax, cm8)
    out_ref[:, 0, :] = vvec
    out_ref[:, 1, :] = xvec
    out_ref[:, 2, :] = yvec


def kernel(confmaps, k):
    b, n, h, w = confmaps.shape
    planes = b * n
    cpb = 32 if planes % 32 == 0 else 1
    x = confmaps.reshape(planes, h, w)
    out = pl.pallas_call(
        _nms_topk_kernel,
        grid=(planes // cpb,),
        in_specs=[pl.BlockSpec((cpb, h, w), lambda i: (i, 0, 0))],
        out_specs=pl.BlockSpec((cpb, 8, 128), lambda i: (i, 0, 0)),
        out_shape=jax.ShapeDtypeStruct((planes, 8, 128), jnp.float32),
    )(x)
    vals = out[:, 0, :_K].reshape(b, n, _K)
    xcoord = out[:, 1, :_K].reshape(b, n, _K)
    ycoord = out[:, 2, :_K].reshape(b, n, _K)
    peaks = jnp.stack([xcoord, ycoord], axis=-1)
    valid = vals > jnp.float32(_THR)
    return peaks, vals, valid
